# baseline (speedup 1.0000x reference)
import functools

import jax
import jax.numpy as jnp
from jax import lax
from jax.experimental import pallas as pl
from jax.experimental.pallas import tpu as pltpu
from jax.experimental.pallas import tpu_sc as plsc

_LANES = 16
_NW = 32


def _sc_body(num_tasks, maxc, rows, x_hbm, ids_hbm, off_hbm, disc_hbm,
             mask_hbm, out_hbm, x_v, ids_v, off_v, disc_v, mask_v, a_v,
             b_v, out_v, sem):
    tbl = num_tasks * maxc
    wid = lax.axis_index("s") * 2 + lax.axis_index("c")
    base = wid * rows
    pltpu.sync_copy(x_hbm.at[pl.ds(base, rows)], x_v)
    pltpu.sync_copy(ids_hbm.at[pl.ds(base, rows)], ids_v)
    pltpu.sync_copy(off_hbm, off_v)
    pltpu.sync_copy(disc_hbm, disc_v)
    pltpu.sync_copy(mask_hbm, mask_v)
    for i in range(tbl // _LANES):
        s = pl.ds(i * _LANES, _LANES)
        p = lax.iota(jnp.int32, _LANES) + i * _LANES
        r = p // maxc
        c = p - r * maxc
        d = (plsc.load_gather(disc_v, [r, c])
             * plsc.load_gather(mask_v, [r, c]))
        a_v[s] = d
        b_v[s] = d * plsc.load_gather(off_v, [r, c])

    groups = rows // _LANES
    nchunks = 4
    gchunk = groups // nchunks
    rchunk = rows // nchunks
    copies = []
    for q in range(nchunks):
        @plsc.parallel_loop(q * gchunk, (q + 1) * gchunk, unroll=4)
        def g_body(g):
            xv = x_v[pl.ds(g * _LANES, _LANES)]
            tbase = ids_v[pl.ds(g * _LANES, _LANES)] * maxc
            c = g * _LANES
            for j in range(maxc):
                av = plsc.load_gather(a_v, [tbase + j])
                bv = plsc.load_gather(b_v, [tbase + j])
                out_v[j, pl.ds(c, _LANES)] = av * xv + bv
        copies.append(pltpu.async_copy(
            out_v.at[:, pl.ds(q * rchunk, rchunk)],
            out_hbm.at[:, pl.ds(base + q * rchunk, rchunk)],
            sem))
    for cp in copies:
        cp.wait()


def kernel(x, offsets, discrimination, mask, task_ids):
    b = x.shape[0]
    num_tasks, maxc = offsets.shape
    tbl = num_tasks * maxc
    rows = b // _NW

    mesh = plsc.VectorSubcoreMesh(core_axis_name="c", subcore_axis_name="s")
    call = pl.kernel(
        functools.partial(_sc_body, num_tasks, maxc, rows),
        out_type=jax.ShapeDtypeStruct((maxc, b), jnp.float32),
        mesh=mesh,
        compiler_params=pltpu.CompilerParams(
            needs_layout_passes=False, use_tc_tiling_on_sc=True),
        scratch_types=[
            pltpu.VMEM((rows,), jnp.float32),
            pltpu.VMEM((rows,), jnp.int32),
            pltpu.VMEM((num_tasks, maxc), jnp.float32),
            pltpu.VMEM((num_tasks, maxc), jnp.float32),
            pltpu.VMEM((num_tasks, maxc), jnp.float32),
            pltpu.VMEM((tbl,), jnp.float32),
            pltpu.VMEM((tbl,), jnp.float32),
            pltpu.VMEM((maxc, rows), jnp.float32),
            pltpu.SemaphoreType.DMA,
        ],
    )
    out_t = call(
        x.reshape(b),
        task_ids.astype(jnp.int32),
        offsets,
        discrimination,
        mask,
    )
    return out_t.T


# speedup vs baseline: 1.1708x; 1.1708x over previous
import functools

import jax
import jax.numpy as jnp
from jax import lax
from jax.experimental import pallas as pl
from jax.experimental.pallas import tpu as pltpu
from jax.experimental.pallas import tpu_sc as plsc

_LANES = 16
_NW = 32


def _sc_body(num_tasks, maxc, rows, x_hbm, ids_hbm, off_hbm, disc_hbm,
             mask_hbm, out_hbm, x_v, ids_v, off_v, disc_v, mask_v, a_v,
             b_v, out_v, sem):
    tbl = num_tasks * maxc
    wid = lax.axis_index("s") * 2 + lax.axis_index("c")
    base = wid * rows
    copies = [
        pltpu.async_copy(x_hbm.at[pl.ds(base, rows)], x_v, sem),
        pltpu.async_copy(ids_hbm.at[pl.ds(base, rows)], ids_v, sem),
        pltpu.async_copy(off_hbm, off_v, sem),
        pltpu.async_copy(disc_hbm, disc_v, sem),
        pltpu.async_copy(mask_hbm, mask_v, sem),
    ]
    for cp in copies:
        cp.wait()
    for i in range(tbl // _LANES):
        s = pl.ds(i * _LANES, _LANES)
        p = lax.iota(jnp.int32, _LANES) + i * _LANES
        r = p // maxc
        c = p - r * maxc
        d = (plsc.load_gather(disc_v, [r, c])
             * plsc.load_gather(mask_v, [r, c]))
        a_v[s] = d
        b_v[s] = d * plsc.load_gather(off_v, [r, c])

    @plsc.parallel_loop(0, rows // _LANES, unroll=4)
    def g_body(g):
        xv = x_v[pl.ds(g * _LANES, _LANES)]
        tbase = ids_v[pl.ds(g * _LANES, _LANES)] * maxc
        c = g * _LANES
        for j in range(maxc):
            av = plsc.load_gather(a_v, [tbase + j])
            bv = plsc.load_gather(b_v, [tbase + j])
            out_v[j, pl.ds(c, _LANES)] = av * xv + bv
    pltpu.sync_copy(out_v, out_hbm.at[:, pl.ds(base, rows)])


def kernel(x, offsets, discrimination, mask, task_ids):
    b = x.shape[0]
    num_tasks, maxc = offsets.shape
    tbl = num_tasks * maxc
    rows = b // _NW

    mesh = plsc.VectorSubcoreMesh(core_axis_name="c", subcore_axis_name="s")
    call = pl.kernel(
        functools.partial(_sc_body, num_tasks, maxc, rows),
        out_type=jax.ShapeDtypeStruct((maxc, b), jnp.float32),
        mesh=mesh,
        compiler_params=pltpu.CompilerParams(
            needs_layout_passes=False, use_tc_tiling_on_sc=True),
        scratch_types=[
            pltpu.VMEM((rows,), jnp.float32),
            pltpu.VMEM((rows,), jnp.int32),
            pltpu.VMEM((num_tasks, maxc), jnp.float32),
            pltpu.VMEM((num_tasks, maxc), jnp.float32),
            pltpu.VMEM((num_tasks, maxc), jnp.float32),
            pltpu.VMEM((tbl,), jnp.float32),
            pltpu.VMEM((tbl,), jnp.float32),
            pltpu.VMEM((maxc, rows), jnp.float32),
            pltpu.SemaphoreType.DMA,
        ],
    )
    out_t = call(
        x.reshape(b),
        task_ids.astype(jnp.int32),
        offsets,
        discrimination,
        mask,
    )
    return out_t.T
